# Initial kernel scaffold; baseline (speedup 1.0000x reference)
import jax
import jax.numpy as jnp
from jax.experimental import pallas as pl


def kernel(user_id, timestamp, user_table, time_table, buckets):
    raise NotImplementedError("write your pallas kernel here")


# speedup vs baseline: 16.4910x; 16.4910x over previous
import functools

import jax
import jax.numpy as jnp
from jax import lax
from jax.experimental import pallas as pl
from jax.experimental.pallas import tpu as pltpu
from jax.experimental.pallas import tpu_sc as plsc


def kernel(user_id, timestamp, user_table, time_table, buckets):
    B = user_id.shape[0]
    UD = user_table.shape[1]
    TD = time_table.shape[1]
    NB = buckets.shape[0]

    info = plsc.get_sparse_core_info()
    NC, NS, L = info.num_cores, info.num_subcores, info.num_lanes
    NW = NC * NS
    bpw = B // NW
    nchunks = bpw // L

    mesh = plsc.VectorSubcoreMesh(core_axis_name="c", subcore_axis_name="s")

    @functools.partial(
        pl.kernel,
        out_type=jax.ShapeDtypeStruct((2 * B, UD), jnp.float32),
        mesh=mesh,
        compiler_params=pltpu.CompilerParams(
            needs_layout_passes=False, use_tc_tiling_on_sc=False
        ),
        scratch_types=[
            pltpu.VMEM((bpw,), jnp.int32),
            pltpu.VMEM((bpw,), jnp.float32),
            pltpu.VMEM((NB,), jnp.float32),
            pltpu.VMEM((bpw,), jnp.int32),
            pltpu.VMEM((bpw,), jnp.int32),
            pltpu.VMEM((bpw,), jnp.int32),
            pltpu.VMEM((bpw, UD), jnp.float32),
            pltpu.VMEM((bpw, TD), jnp.float32),
            pltpu.SemaphoreType.DMA,
            pltpu.SemaphoreType.DMA,
            pltpu.SemaphoreType.DMA,
            pltpu.SemaphoreType.DMA,
        ],
    )
    def body(uid_hbm, ts_hbm, utab_hbm, ttab_hbm, bkt_hbm, out_hbm,
             uidx_v, ts_v, bkt_v, tidx_v, srow_u_v, srow_t_v,
             urows_v, trows_v, sem_u, sem_t, sem_ou, sem_ot):
        wid = lax.axis_index("s") * NC + lax.axis_index("c")
        base = wid * bpw

        pltpu.sync_copy(uid_hbm.at[pl.ds(base, bpw)], uidx_v)
        ucopy = pltpu.async_copy(utab_hbm.at[uidx_v], urows_v, sem_u)

        pltpu.sync_copy(bkt_hbm, bkt_v)
        pltpu.sync_copy(ts_hbm.at[pl.ds(base, bpw)], ts_v)

        def chunk(c, carry):
            v = ts_v[pl.ds(c * L, L)]
            ans = jnp.zeros((L,), jnp.int32)
            k = NB
            while k >= 1:
                probe = jnp.minimum(ans + (k - 1), NB - 1)
                bv = plsc.load_gather(bkt_v, [probe])
                pred = (bv <= v) & (ans + k <= NB)
                ans = jnp.where(pred, ans + k, ans)
                k //= 2
            tidx_v[pl.ds(c * L, L)] = ans
            srow = (base + c * L) * 2 + lax.iota(jnp.int32, L) * 2
            srow_u_v[pl.ds(c * L, L)] = srow
            srow_t_v[pl.ds(c * L, L)] = srow + 1
            return carry

        lax.fori_loop(0, nchunks, chunk, 0)

        tcopy = pltpu.async_copy(ttab_hbm.at[tidx_v], trows_v, sem_t)
        ucopy.wait()
        oucopy = pltpu.async_copy(urows_v, out_hbm.at[srow_u_v], sem_ou)
        tcopy.wait()
        otcopy = pltpu.async_copy(trows_v, out_hbm.at[srow_t_v], sem_ot)
        oucopy.wait()
        otcopy.wait()

    out2 = body(user_id, timestamp, user_table, time_table, buckets)
    return out2.reshape(B, UD + TD)
